# Initial kernel scaffold; baseline (speedup 1.0000x reference)
import jax
import jax.numpy as jnp
from jax.experimental import pallas as pl


def kernel(vectors, node_feats, radial_embedding, senders, receivers, W0, W1, W2, W3):
    raise NotImplementedError("write your pallas kernel here")


# speedup vs baseline: 18.1287x; 18.1287x over previous
import functools
import math

import jax
import jax.numpy as jnp
from jax import lax
from jax.experimental import pallas as pl
from jax.experimental.pallas import tpu as pltpu
from jax.experimental.pallas import tpu_sc as plsc

_N = 10000
_NPAD = 10240
_E = 160000
_D = 128
_ACT_NORM = 0.5595
_AVG = 32.0

_NC, _NS = 2, 16
_NW = _NC * _NS

_W = 40
_G_WPW = _E // _W // _NW
_S_WPT = _E // _W // _NS
_N_PT = _NPAD // _NS

_BE = 4000
_TC_GRID = _E // _BE


def _gather_body(node_hbm, send_hbm, sg_hbm, sidx_v, gbuf, sem):
    cid = lax.axis_index("c")
    sid = lax.axis_index("s")
    wid = sid * _NC + cid
    pltpu.sync_copy(send_hbm.at[wid], sidx_v)

    def win(w, carry):
        pltpu.async_copy(node_hbm.at[sidx_v.at[w]], gbuf, sem).wait()
        pltpu.sync_copy(gbuf, sg_hbm.at[pl.ds((wid * _G_WPW + w) * _W, _W)])
        return carry

    lax.fori_loop(0, _G_WPW, win, 0)


def _sc_gather(node_feats, send3d):
    mesh = plsc.VectorSubcoreMesh(core_axis_name="c", subcore_axis_name="s")
    f = functools.partial(
        pl.kernel,
        out_type=jax.ShapeDtypeStruct((_E, _D), jnp.float32),
        mesh=mesh,
        scratch_types=[
            pltpu.VMEM((_G_WPW, _W), jnp.int32),
            pltpu.VMEM((_W, _D), jnp.float32),
            pltpu.SemaphoreType.DMA,
        ],
    )(_gather_body)
    return f(node_feats, send3d)


def _tc_body(rad_ref, vec_ref, sg_ref, w0, w1, w2, w3, out_ref):
    rad = rad_ref[...]
    h = jnp.dot(rad, w0[...], preferred_element_type=jnp.float32)
    h = jax.nn.silu(h) * (1.0 / _ACT_NORM)
    h = jnp.dot(h, w1[...], preferred_element_type=jnp.float32)
    h = jax.nn.silu(h) * (1.0 / _ACT_NORM)
    h = jnp.dot(h, w2[...], preferred_element_type=jnp.float32)
    h = jax.nn.silu(h) * (1.0 / _ACT_NORM)
    mix = jnp.dot(h, w3[...], preferred_element_type=jnp.float32)

    v = -vec_ref[...]
    nrm = jnp.sqrt(jnp.sum(v * v, axis=-1, keepdims=True))
    sh = (math.sqrt(3.0) / _AVG) * v / (nrm + 1e-9)

    g = sg_ref[...]
    a = g * mix[:, :_D] * (1.0 / _AVG)
    b = g * mix[:, _D:]
    out_ref[0] = a
    out_ref[1] = b * sh[:, 0:1]
    out_ref[2] = b * sh[:, 1:2]
    out_ref[3] = b * sh[:, 2:3]


def _tc_messages(radial, vectors, sg, W0s, W1s, W2s, W3s):
    return pl.pallas_call(
        _tc_body,
        grid=(_TC_GRID,),
        in_specs=[
            pl.BlockSpec((_BE, 8), lambda i: (i, 0)),
            pl.BlockSpec((_BE, 3), lambda i: (i, 0)),
            pl.BlockSpec((_BE, _D), lambda i: (i, 0)),
            pl.BlockSpec((8, 64), lambda i: (0, 0)),
            pl.BlockSpec((64, 64), lambda i: (0, 0)),
            pl.BlockSpec((64, 64), lambda i: (0, 0)),
            pl.BlockSpec((64, 2 * _D), lambda i: (0, 0)),
        ],
        out_specs=pl.BlockSpec((4, _BE, _D), lambda i: (0, i, 0)),
        out_shape=jax.ShapeDtypeStruct((4, _E, _D), jnp.float32),
    )(radial, vectors, sg, W0s, W1s, W2s, W3s)


def _scatter_body(msgs_hbm, recv_hbm, out_hbm, ridx_v, upd_v, acc, sem):
    cid = lax.axis_index("c")
    sid = lax.axis_index("s")
    pltpu.sync_copy(recv_hbm.at[sid], ridx_v)

    for cc in range(2):
        chunk = cid * 2 + cc

        def zb(i, carry):
            upd_v[i // 8, pl.ds((i % 8) * 16, 16)] = jnp.zeros((16,), jnp.float32)
            return carry

        lax.fori_loop(0, _W * (_D // 16), zb, 0)

        def zc(j, carry):
            pltpu.sync_copy(upd_v, acc.at[pl.ds(sid * _N_PT + j * _W, _W)])
            return carry

        lax.fori_loop(0, _N_PT // _W, zc, 0)
        plsc.subcore_barrier()

        def win(w, carry):
            e0 = (sid * _S_WPT + w) * _W
            pltpu.sync_copy(msgs_hbm.at[chunk, pl.ds(e0, _W)], upd_v)
            pltpu.sync_copy(upd_v, acc.at[ridx_v.at[w]], add=True)
            return carry

        lax.fori_loop(0, _S_WPT, win, 0)
        plsc.subcore_barrier()
        pltpu.sync_copy(
            acc.at[pl.ds(sid * _N_PT, _N_PT)],
            out_hbm.at[chunk, pl.ds(sid * _N_PT, _N_PT)],
        )
        plsc.subcore_barrier()


def _sc_scatter(msgs, recv3d):
    mesh = plsc.VectorSubcoreMesh(core_axis_name="c", subcore_axis_name="s")
    f = functools.partial(
        pl.kernel,
        out_type=jax.ShapeDtypeStruct((4, _NPAD, _D), jnp.float32),
        mesh=mesh,
        scratch_types=[
            pltpu.VMEM((_S_WPT, _W), jnp.int32),
            pltpu.VMEM((_W, _D), jnp.float32),
            pltpu.VMEM_SHARED((_NPAD, _D), jnp.float32),
            pltpu.SemaphoreType.DMA,
        ],
    )(_scatter_body)
    return f(msgs, recv3d)


def kernel(vectors, node_feats, radial_embedding, senders, receivers, W0, W1, W2, W3):
    send3d = senders.reshape(_NW, _G_WPW, _W)
    recv3d = receivers.reshape(_NS, _S_WPT, _W)
    W0s = W0 * (1.0 / math.sqrt(W0.shape[0]))
    W1s = W1 * (1.0 / math.sqrt(W1.shape[0]))
    W2s = W2 * (1.0 / math.sqrt(W2.shape[0]))
    W3s = W3 * (1.0 / math.sqrt(W3.shape[0]))

    sg = _sc_gather(node_feats, send3d)
    msgs = _tc_messages(radial_embedding, vectors, sg, W0s, W1s, W2s, W3s)
    out4 = _sc_scatter(msgs, recv3d)[:, :_N]

    out_s = out4[0]
    out_v = jnp.transpose(out4[1:], (1, 2, 0)).reshape(_N, 3 * _D)
    return jnp.concatenate([out_s, out_v], axis=-1)


# speedup vs baseline: 23.9939x; 1.3235x over previous
import functools
import math

import jax
import jax.numpy as jnp
from jax import lax
from jax.experimental import pallas as pl
from jax.experimental.pallas import tpu as pltpu
from jax.experimental.pallas import tpu_sc as plsc

_N = 10000
_NPAD = 10240
_E = 160000
_D = 128
_ACT_NORM = 0.5595
_AVG = 32.0

_NC, _NS = 2, 16
_NW = _NC * _NS

_GW = 40
_G_WPW = _E // _GW // _NW
_SW = 40
_S_WPT = _E // _SW // _NS
_N_PT = _NPAD // _NS
_NBUF = 5
_SNBUF = 2

_BE = 4000
_TC_GRID = _E // _BE


def _gather_body(node_hbm, send_hbm, sg_hbm, sidx_v, gbuf, sem_in, sem_out):
    cid = lax.axis_index("c")
    sid = lax.axis_index("s")
    wid = sid * _NC + cid
    pltpu.sync_copy(send_hbm.at[wid], sidx_v)

    @pl.loop(0, _G_WPW, step=_NBUF)
    def grp(w0):
        for b in range(_NBUF):
            @pl.when(w0 > 0)
            def _():
                pltpu.make_async_copy(
                    gbuf.at[b],
                    sg_hbm.at[pl.ds((wid * _G_WPW) * _GW, _GW)],
                    sem_out.at[b],
                ).wait()

            pltpu.async_copy(node_hbm.at[sidx_v.at[w0 + b]], gbuf.at[b], sem_in.at[b])
        for b in range(_NBUF):
            pltpu.make_async_copy(
                node_hbm.at[sidx_v.at[w0 + b]], gbuf.at[b], sem_in.at[b]
            ).wait()
            pltpu.async_copy(
                gbuf.at[b],
                sg_hbm.at[pl.ds((wid * _G_WPW + w0 + b) * _GW, _GW)],
                sem_out.at[b],
            )

    for b in range(_NBUF):
        pltpu.make_async_copy(
            gbuf.at[b],
            sg_hbm.at[pl.ds((wid * _G_WPW) * _GW, _GW)],
            sem_out.at[b],
        ).wait()


def _sc_gather(node_feats, send3d):
    mesh = plsc.VectorSubcoreMesh(core_axis_name="c", subcore_axis_name="s")
    f = functools.partial(
        pl.kernel,
        out_type=jax.ShapeDtypeStruct((_E, _D), jnp.float32),
        mesh=mesh,
        scratch_types=[
            pltpu.VMEM((_G_WPW, _GW), jnp.int32),
            pltpu.VMEM((_NBUF, _GW, _D), jnp.float32),
            pltpu.SemaphoreType.DMA((_NBUF,)),
            pltpu.SemaphoreType.DMA((_NBUF,)),
        ],
    )(_gather_body)
    return f(node_feats, send3d)


def _tc_body(rad_ref, vec_ref, sg_ref, w0, w1, w2, w3, out_ref):
    rad = rad_ref[...]
    h = jnp.dot(rad, w0[...], preferred_element_type=jnp.float32)
    h = jax.nn.silu(h) * (1.0 / _ACT_NORM)
    h = jnp.dot(h, w1[...], preferred_element_type=jnp.float32)
    h = jax.nn.silu(h) * (1.0 / _ACT_NORM)
    h = jnp.dot(h, w2[...], preferred_element_type=jnp.float32)
    h = jax.nn.silu(h) * (1.0 / _ACT_NORM)
    mix = jnp.dot(h, w3[...], preferred_element_type=jnp.float32)

    v = -vec_ref[...]
    nrm = jnp.sqrt(jnp.sum(v * v, axis=-1, keepdims=True))
    sh = (math.sqrt(3.0) / _AVG) * v / (nrm + 1e-9)

    g = sg_ref[...]
    a = g * mix[:, :_D] * (1.0 / _AVG)
    b = g * mix[:, _D:]
    out_ref[0] = a
    out_ref[1] = b * sh[:, 0:1]
    out_ref[2] = b * sh[:, 1:2]
    out_ref[3] = b * sh[:, 2:3]


def _tc_messages(radial, vectors, sg, W0s, W1s, W2s, W3s):
    return pl.pallas_call(
        _tc_body,
        grid=(_TC_GRID,),
        in_specs=[
            pl.BlockSpec((_BE, 8), lambda i: (i, 0)),
            pl.BlockSpec((_BE, 3), lambda i: (i, 0)),
            pl.BlockSpec((_BE, _D), lambda i: (i, 0)),
            pl.BlockSpec((8, 64), lambda i: (0, 0)),
            pl.BlockSpec((64, 64), lambda i: (0, 0)),
            pl.BlockSpec((64, 64), lambda i: (0, 0)),
            pl.BlockSpec((64, 2 * _D), lambda i: (0, 0)),
        ],
        out_specs=pl.BlockSpec((4, _BE, _D), lambda i: (0, i, 0)),
        out_shape=jax.ShapeDtypeStruct((4, _E, _D), jnp.float32),
    )(radial, vectors, sg, W0s, W1s, W2s, W3s)


def _scatter_body(msgs_hbm, recv_hbm, out_hbm, ridx_v, bufs, acc, sem_in, sem_out):
    cid = lax.axis_index("c")
    sid = lax.axis_index("s")
    pltpu.sync_copy(recv_hbm.at[sid], ridx_v)

    for cc in range(2):
        chunk = cid * 2 + cc

        def zb(i, carry):
            bufs[0, i // (_D // 16), pl.ds((i % (_D // 16)) * 16, 16)] = jnp.zeros(
                (16,), jnp.float32
            )
            return carry

        lax.fori_loop(0, _SW * (_D // 16), zb, 0)

        def zc(j, carry):
            pltpu.sync_copy(bufs.at[0], acc.at[pl.ds(sid * _N_PT + j * _SW, _SW)])
            return carry

        lax.fori_loop(0, _N_PT // _SW, zc, 0)
        plsc.subcore_barrier()

        @pl.loop(0, _S_WPT, step=_SNBUF)
        def grp(w0):
            for b in range(_SNBUF):
                @pl.when(w0 > 0)
                def _():
                    pltpu.make_async_copy(
                        bufs.at[b], acc.at[ridx_v.at[0]], sem_out.at[b]
                    ).wait()

                e0 = sid * (_S_WPT * _SW) + (w0 + b) * _SW
                pltpu.async_copy(
                    msgs_hbm.at[chunk, pl.ds(e0, _SW)], bufs.at[b], sem_in.at[b]
                )
            for b in range(_SNBUF):
                e0 = sid * (_S_WPT * _SW) + (w0 + b) * _SW
                pltpu.make_async_copy(
                    msgs_hbm.at[chunk, pl.ds(e0, _SW)], bufs.at[b], sem_in.at[b]
                ).wait()
                pltpu.async_copy(
                    bufs.at[b], acc.at[ridx_v.at[w0 + b]], sem_out.at[b], add=True
                )

        for b in range(_SNBUF):
            pltpu.make_async_copy(
                bufs.at[b], acc.at[ridx_v.at[0]], sem_out.at[b]
            ).wait()

        plsc.subcore_barrier()
        pltpu.sync_copy(
            acc.at[pl.ds(sid * _N_PT, _N_PT)],
            out_hbm.at[chunk, pl.ds(sid * _N_PT, _N_PT)],
        )
        plsc.subcore_barrier()


def _sc_scatter(msgs, recv3d):
    mesh = plsc.VectorSubcoreMesh(core_axis_name="c", subcore_axis_name="s")
    f = functools.partial(
        pl.kernel,
        out_type=jax.ShapeDtypeStruct((4, _NPAD, _D), jnp.float32),
        mesh=mesh,
        scratch_types=[
            pltpu.VMEM((_S_WPT, _SW), jnp.int32),
            pltpu.VMEM((_SNBUF, _SW, _D), jnp.float32),
            pltpu.VMEM_SHARED((_NPAD, _D), jnp.float32),
            pltpu.SemaphoreType.DMA((_SNBUF,)),
            pltpu.SemaphoreType.DMA((_SNBUF,)),
        ],
    )(_scatter_body)
    return f(msgs, recv3d)


def kernel(vectors, node_feats, radial_embedding, senders, receivers, W0, W1, W2, W3):
    send3d = senders.reshape(_NW, _G_WPW, _GW)
    recv3d = receivers.reshape(_NS, _S_WPT, _SW)
    W0s = W0 * (1.0 / math.sqrt(W0.shape[0]))
    W1s = W1 * (1.0 / math.sqrt(W1.shape[0]))
    W2s = W2 * (1.0 / math.sqrt(W2.shape[0]))
    W3s = W3 * (1.0 / math.sqrt(W3.shape[0]))

    sg = _sc_gather(node_feats, send3d)
    msgs = _tc_messages(radial_embedding, vectors, sg, W0s, W1s, W2s, W3s)
    out4 = _sc_scatter(msgs, recv3d)[:, :_N]

    out_s = out4[0]
    out_v = jnp.transpose(out4[1:], (1, 2, 0)).reshape(_N, 3 * _D)
    return jnp.concatenate([out_s, out_v], axis=-1)


# speedup vs baseline: 30.6807x; 1.2787x over previous
import functools
import math

import jax
import jax.numpy as jnp
from jax import lax
from jax.experimental import pallas as pl
from jax.experimental.pallas import tpu as pltpu
from jax.experimental.pallas import tpu_sc as plsc

_N = 10000
_NPAD = 10112
_E = 160000
_D = 128
_ACT_NORM = 0.5595
_AVG = 32.0

_NC, _NS = 2, 16
_NW = _NC * _NS

_GW = 40
_G_WPW = _E // _GW // _NW
_SW = 40
_S_WPT = _E // _SW // _NS
_N_PT = _NPAD // _NS
_NBUF = 5
_SNBUF = 5
_S_NG = _S_WPT // _SNBUF

_BE = 4000
_TC_GRID = _E // _BE


def _gather_body(node_hbm, send_hbm, sg_hbm, sidx_v, gbuf, nodes_s, sem_in, sem_out):
    cid = lax.axis_index("c")
    sid = lax.axis_index("s")
    wid = sid * _NC + cid

    @pl.when(sid < _NS - 1)
    def _():
        pltpu.sync_copy(
            node_hbm.at[pl.ds(sid * 632, 632)], nodes_s.at[pl.ds(sid * 632, 632)]
        )

    @pl.when(sid == _NS - 1)
    def _():
        pltpu.sync_copy(
            node_hbm.at[pl.ds(9480, _N - 9480)], nodes_s.at[pl.ds(9480, _N - 9480)]
        )

    pltpu.sync_copy(send_hbm.at[wid], sidx_v)
    plsc.subcore_barrier()

    @pl.loop(0, _G_WPW, step=_NBUF)
    def grp(w0):
        for b in range(_NBUF):
            @pl.when(w0 > 0)
            def _():
                pltpu.make_async_copy(
                    gbuf.at[b],
                    sg_hbm.at[pl.ds((wid * _G_WPW) * _GW, _GW)],
                    sem_out.at[b],
                ).wait()

            pltpu.async_copy(nodes_s.at[sidx_v.at[w0 + b]], gbuf.at[b], sem_in.at[b])
        for b in range(_NBUF):
            pltpu.make_async_copy(
                nodes_s.at[sidx_v.at[w0 + b]], gbuf.at[b], sem_in.at[b]
            ).wait()
            pltpu.async_copy(
                gbuf.at[b],
                sg_hbm.at[pl.ds((wid * _G_WPW + w0 + b) * _GW, _GW)],
                sem_out.at[b],
            )

    for b in range(_NBUF):
        pltpu.make_async_copy(
            gbuf.at[b],
            sg_hbm.at[pl.ds((wid * _G_WPW) * _GW, _GW)],
            sem_out.at[b],
        ).wait()


def _sc_gather(node_feats, send3d):
    mesh = plsc.VectorSubcoreMesh(core_axis_name="c", subcore_axis_name="s")
    f = functools.partial(
        pl.kernel,
        out_type=jax.ShapeDtypeStruct((_E, _D), jnp.float32),
        mesh=mesh,
        scratch_types=[
            pltpu.VMEM((_G_WPW, _GW), jnp.int32),
            pltpu.VMEM((_NBUF, _GW, _D), jnp.float32),
            pltpu.VMEM_SHARED((_N, _D), jnp.float32),
            pltpu.SemaphoreType.DMA((_NBUF,)),
            pltpu.SemaphoreType.DMA((_NBUF,)),
        ],
    )(_gather_body)
    return f(node_feats, send3d)


def _tc_body(rad_ref, vec_ref, sg_ref, w0, w1, w2, w3, out_ref):
    rad = rad_ref[...]
    h = jnp.dot(rad, w0[...], preferred_element_type=jnp.float32)
    h = jax.nn.silu(h) * (1.0 / _ACT_NORM)
    h = jnp.dot(h, w1[...], preferred_element_type=jnp.float32)
    h = jax.nn.silu(h) * (1.0 / _ACT_NORM)
    h = jnp.dot(h, w2[...], preferred_element_type=jnp.float32)
    h = jax.nn.silu(h) * (1.0 / _ACT_NORM)
    mix = jnp.dot(h, w3[...], preferred_element_type=jnp.float32)

    v = -vec_ref[...]
    nrm = jnp.sqrt(jnp.sum(v * v, axis=-1, keepdims=True))
    sh = (math.sqrt(3.0) / _AVG) * v / (nrm + 1e-9)

    g = sg_ref[...]
    a = g * mix[:, :_D] * (1.0 / _AVG)
    b = g * mix[:, _D:]
    out_ref[0] = a
    out_ref[1] = b * sh[:, 0:1]
    out_ref[2] = b * sh[:, 1:2]
    out_ref[3] = b * sh[:, 2:3]


def _tc_messages(radial, vectors, sg, W0s, W1s, W2s, W3s):
    return pl.pallas_call(
        _tc_body,
        grid=(_TC_GRID,),
        in_specs=[
            pl.BlockSpec((_BE, 8), lambda i: (i, 0)),
            pl.BlockSpec((_BE, 3), lambda i: (i, 0)),
            pl.BlockSpec((_BE, _D), lambda i: (i, 0)),
            pl.BlockSpec((8, 64), lambda i: (0, 0)),
            pl.BlockSpec((64, 64), lambda i: (0, 0)),
            pl.BlockSpec((64, 64), lambda i: (0, 0)),
            pl.BlockSpec((64, 2 * _D), lambda i: (0, 0)),
        ],
        out_specs=pl.BlockSpec((4, _BE, _D), lambda i: (0, i, 0)),
        out_shape=jax.ShapeDtypeStruct((4, _E, _D), jnp.float32),
    )(radial, vectors, sg, W0s, W1s, W2s, W3s)


def _scatter_body(msgs_hbm, recv_hbm, out_hbm, rbufs, bufs, acc, sem_in, sem_out, rsem):
    cid = lax.axis_index("c")
    sid = lax.axis_index("s")

    for cc in range(2):
        chunk = cid * 2 + cc

        def zb(i, carry):
            bufs[0, i // (_D // 16), pl.ds((i % (_D // 16)) * 16, 16)] = jnp.zeros(
                (16,), jnp.float32
            )
            return carry

        lax.fori_loop(0, _SW * (_D // 16), zb, 0)

        def zc(j, carry):
            pltpu.sync_copy(bufs.at[0], acc.at[pl.ds(sid * _N_PT + j * _SW, _SW)])
            return carry

        lax.fori_loop(0, _N_PT // _SW, zc, 0)
        pltpu.sync_copy(
            bufs.at[0, pl.ds(0, _N_PT % _SW)],
            acc.at[pl.ds(sid * _N_PT + (_N_PT // _SW) * _SW, _N_PT % _SW)],
        )
        plsc.subcore_barrier()

        pltpu.sync_copy(recv_hbm.at[sid, 0], rbufs.at[0])

        @pl.loop(0, _S_NG, step=2)
        def grp2(g0):
            for gg in range(2):
                g = g0 + gg

                @pl.when(g > 0)
                def _():
                    pltpu.make_async_copy(
                        recv_hbm.at[sid, 0], rbufs.at[gg], rsem.at[gg]
                    ).wait()

                for b in range(_SNBUF):
                    @pl.when(g > 0)
                    def _():
                        pltpu.make_async_copy(
                            bufs.at[b], acc.at[rbufs.at[0, 0]], sem_out.at[b]
                        ).wait()

                    e0 = sid * (_S_WPT * _SW) + (g * _SNBUF + b) * _SW
                    pltpu.async_copy(
                        msgs_hbm.at[chunk, pl.ds(e0, _SW)], bufs.at[b], sem_in.at[b]
                    )

                @pl.when(g + 1 < _S_NG)
                def _():
                    pltpu.async_copy(
                        recv_hbm.at[sid, g + 1], rbufs.at[1 - gg], rsem.at[1 - gg]
                    )

                for b in range(_SNBUF):
                    e0 = sid * (_S_WPT * _SW) + (g * _SNBUF + b) * _SW
                    pltpu.make_async_copy(
                        msgs_hbm.at[chunk, pl.ds(e0, _SW)], bufs.at[b], sem_in.at[b]
                    ).wait()
                    pltpu.async_copy(
                        bufs.at[b], acc.at[rbufs.at[gg, b]], sem_out.at[b], add=True
                    )

        for b in range(_SNBUF):
            pltpu.make_async_copy(
                bufs.at[b], acc.at[rbufs.at[0, 0]], sem_out.at[b]
            ).wait()

        plsc.subcore_barrier()
        pltpu.sync_copy(
            acc.at[pl.ds(sid * _N_PT, _N_PT)],
            out_hbm.at[chunk, pl.ds(sid * _N_PT, _N_PT)],
        )
        plsc.subcore_barrier()


def _sc_scatter(msgs, recv4d):
    mesh = plsc.VectorSubcoreMesh(core_axis_name="c", subcore_axis_name="s")
    f = functools.partial(
        pl.kernel,
        out_type=jax.ShapeDtypeStruct((4, _NPAD, _D), jnp.float32),
        mesh=mesh,
        scratch_types=[
            pltpu.VMEM((2, _SNBUF, _SW), jnp.int32),
            pltpu.VMEM((_SNBUF, _SW, _D), jnp.float32),
            pltpu.VMEM_SHARED((_NPAD, _D), jnp.float32),
            pltpu.SemaphoreType.DMA((_SNBUF,)),
            pltpu.SemaphoreType.DMA((_SNBUF,)),
            pltpu.SemaphoreType.DMA((2,)),
        ],
    )(_scatter_body)
    return f(msgs, recv4d)


def kernel(vectors, node_feats, radial_embedding, senders, receivers, W0, W1, W2, W3):
    send3d = senders.reshape(_NW, _G_WPW, _GW)
    recv4d = receivers.reshape(_NS, _S_NG, _SNBUF, _SW)
    W0s = W0 * (1.0 / math.sqrt(W0.shape[0]))
    W1s = W1 * (1.0 / math.sqrt(W1.shape[0]))
    W2s = W2 * (1.0 / math.sqrt(W2.shape[0]))
    W3s = W3 * (1.0 / math.sqrt(W3.shape[0]))

    sg = _sc_gather(node_feats, send3d)
    msgs = _tc_messages(radial_embedding, vectors, sg, W0s, W1s, W2s, W3s)
    out4 = _sc_scatter(msgs, recv4d)[:, :_N]

    out_s = out4[0]
    out_v = jnp.transpose(out4[1:], (1, 2, 0)).reshape(_N, 3 * _D)
    return jnp.concatenate([out_s, out_v], axis=-1)
